# baseline (speedup 1.0000x reference)
import functools

import jax
import jax.numpy as jnp
from jax import lax
from jax.experimental import pallas as pl
from jax.experimental.pallas import tpu as pltpu
from jax.experimental.pallas import tpu_sc as plsc

BATCH = 16384
NROWS = 1000000
D = 64
NC = 2
NS = 16
NW = NC * NS
BPW = BATCH // NW
NBLK = 7813
NFULL = 7812
RPW = 246
CW = 128
LCAP = 768
SCAP = 144
MCAP = 32
IDCH = BATCH // 16
CH = 128
NCH = BPW // CH
GROUPS = BPW // 16

_params = pltpu.CompilerParams(
    needs_layout_passes=False, use_tc_tiling_on_sc=True)
_mesh = lambda: plsc.VectorSubcoreMesh(core_axis_name="c", subcore_axis_name="s")


def _gather_body(uid_hbm, tid_hbm, uembT_hbm, tembT_hbm, utail_hbm,
                 ttail_hbm, ustage_hbm, tstage_hbm,
                 ids_v, ulist_id, ulist_pos, tlist_id, tlist_pos,
                 usub_id, usub_pos, tsub_id, tsub_pos,
                 ubuf, tbuf, utail, ttail, mb_id, mb_pos, uring, tring,
                 subcnt, fsem, ssem):
  wid = lax.axis_index("s") * NC + lax.axis_index("c")
  lo_b = wid * RPW
  hi_b = jnp.minimum(lo_b + RPW, NFULL)
  nchunks = hi_b - lo_b

  lanes = lax.iota(jnp.int32, 16)

  def bin_ids(ids_hbm, list_id, list_pos):
    cnt = jnp.int32(0)
    for h in range(4):
      pltpu.sync_copy(ids_hbm.at[pl.ds(h * (BATCH // 4), BATCH // 4)], ids_v)

      def body(k, cnt):
        idv = ids_v[pl.ds(k * 16, 16)]
        blk = lax.shift_right_logical(idv, 7)
        m = jnp.logical_and(blk >= lo_b, blk < lo_b + RPW)
        plsc.store_compressed(list_id.at[pl.ds(cnt, 16)], idv, mask=m)
        plsc.store_compressed(list_pos.at[pl.ds(cnt, 16)],
                              h * (BATCH // 4) + k * 16 + lanes, mask=m)
        n = plsc.all_reduce_population_count(m)
        return cnt + n[0]

      cnt = lax.fori_loop(0, IDCH // 4, body, cnt)
    return cnt

  ucnt = bin_ids(uid_hbm, ulist_id, ulist_pos)
  tcnt = bin_ids(tid_hbm, tlist_id, tlist_pos)

  def split_list(list_id, list_pos, sub_id, sub_pos, cnt, cbase):
    ngroups = lax.shift_right_logical(cnt + 15, 4)
    for sub in range(8):
      def body(g, scnt):
        gsl = pl.ds(g * 16, 16)
        idv = list_id[gsl]
        posv = list_pos[gsl]
        chunk = lax.shift_right_logical(idv, 7) - lo_b
        m = jnp.logical_and(lax.shift_right_logical(chunk, 5) == sub,
                            g * 16 + lanes < cnt)
        plsc.store_compressed(sub_id.at[sub, pl.ds(scnt, 16)], idv, mask=m)
        plsc.store_compressed(sub_pos.at[sub, pl.ds(scnt, 16)], posv, mask=m)
        n = plsc.all_reduce_population_count(m)
        return scnt + n[0]
      scnt = lax.fori_loop(0, ngroups, body, jnp.int32(0))
      subcnt[cbase + sub] = jnp.minimum(scnt, SCAP - 16)

  split_list(ulist_id, ulist_pos, usub_id, usub_pos, ucnt, 0)
  split_list(tlist_id, tlist_pos, tsub_id, tsub_pos, tcnt, 8)

  crange = [lanes + 16 * k for k in range(4)]

  def drain_n(n, ring):
    def dbody(j, _):
      pltpu.make_async_copy(ustage_hbm.at[pl.ds(0, D)], ring.at[0, 0],
                            ssem).wait()
      return ()
    lax.fori_loop(0, n, dbody, ())

  def process(i, cb, buf, sub_id, sub_pos, cbase, ring, bank, pslot,
              stage_hbm):
    sub = lax.shift_right_logical(i, 5)
    scnt = subcnt[cbase + sub]
    ngroups = lax.shift_right_logical(scnt + 15, 4)

    drain_n(subcnt[bank + pslot], ring)

    def scan(g, mcnt):
      gsl = pl.ds(g * 16, 16)
      idv = sub_id[sub, gsl]
      posv = sub_pos[sub, gsl]
      chunk = lax.shift_right_logical(idv, 7) - lo_b
      m = jnp.logical_and(chunk == i, g * 16 + lanes < scnt)
      plsc.store_compressed(mb_id.at[pl.ds(mcnt, 16)], idv, mask=m)
      plsc.store_compressed(mb_pos.at[pl.ds(mcnt, 16)], posv, mask=m)
      n = plsc.all_reduce_population_count(m)
      return mcnt + n[0]

    mcnt = lax.fori_loop(0, ngroups, scan, jnp.int32(0))
    mcnt = jnp.minimum(mcnt, MCAP)

    def extract(j, _):
      idj = mb_id[pl.ds(j, 16)][0]
      posj = mb_pos[pl.ds(j, 16)][0]
      colw = jnp.broadcast_to(idj - cb, (16,))
      slot = jnp.bitwise_and(j, MCAP - 1)
      for k in range(4):
        ring[pslot, slot, pl.ds(k * 16, 16)] = plsc.load_gather(
            buf, [crange[k], colw])
      pltpu.async_copy(ring.at[pslot, slot],
                       stage_hbm.at[pl.ds(posj * D, D)], ssem)
      return ()

    lax.fori_loop(0, mcnt, extract, ())
    subcnt[bank + pslot] = mcnt

  def fire(i, slot):
    off = pl.multiple_of((lo_b + i) * 128, 128)
    pltpu.async_copy(uembT_hbm.at[:, pl.ds(off, CW)], ubuf.at[slot], fsem)
    pltpu.async_copy(tembT_hbm.at[:, pl.ds(off, CW)], tbuf.at[slot], fsem)

  def wait_pair():
    pltpu.make_async_copy(uembT_hbm.at[:, pl.ds(0, CW)], ubuf.at[0],
                          fsem).wait()
    pltpu.make_async_copy(tembT_hbm.at[:, pl.ds(0, CW)], tbuf.at[0],
                          fsem).wait()

  for pre in range(3):
    fire(jnp.int32(pre), pre)

  def chunk_body(i, _):
    @pl.when(i < nchunks)
    def _():
      fire(i, jnp.bitwise_and(i, 3))

    wait_pair()
    pi = i - 3
    cb = (lo_b + pi) * 128
    pslot = jnp.bitwise_and(pi, 3)
    process(pi, cb, ubuf.at[pslot], usub_id, usub_pos, 0, uring, 16, pslot,
            ustage_hbm)
    process(pi, cb, tbuf.at[pslot], tsub_id, tsub_pos, 8, tring, 20, pslot,
            tstage_hbm)
    return ()

  for z in range(8):
    subcnt[16 + z] = jnp.int32(0)

  lax.fori_loop(3, nchunks + 3, chunk_body, ())

  for z in range(4):
    drain_n(subcnt[16 + z], uring)
    drain_n(subcnt[20 + z], tring)

  @pl.when(wid == NW - 1)
  def _():
    pltpu.async_copy(utail_hbm, utail, fsem)
    pltpu.async_copy(ttail_hbm, ttail, fsem)
    pltpu.make_async_copy(utail_hbm, utail, fsem).wait()
    pltpu.make_async_copy(ttail_hbm, ttail, fsem).wait()

    def tail_table(list_id, list_pos, cnt, buf, stage_hbm):
      ngroups = lax.shift_right_logical(cnt + 15, 4)

      def scan(g, mcnt):
        gsl = pl.ds(g * 16, 16)
        idv = list_id[gsl]
        posv = list_pos[gsl]
        m = jnp.logical_and(lax.shift_right_logical(idv, 7) == NFULL,
                            g * 16 + lanes < cnt)
        plsc.store_compressed(mb_id.at[pl.ds(mcnt, 16)], idv, mask=m)
        plsc.store_compressed(mb_pos.at[pl.ds(mcnt, 16)], posv, mask=m)
        n = plsc.all_reduce_population_count(m)
        return mcnt + n[0]

      mcnt = jnp.minimum(lax.fori_loop(0, ngroups, scan, jnp.int32(0)), MCAP)

      def extract(j, _):
        idj = mb_id[pl.ds(j, 16)][0]
        posj = mb_pos[pl.ds(j, 16)][0]
        colw = jnp.broadcast_to(jnp.bitwise_and(idj, 127), (16,))
        slot = jnp.bitwise_and(j, MCAP - 1)
        for k in range(4):
          uring[0, slot, pl.ds(k * 16, 16)] = plsc.load_gather(
              buf, [crange[k], colw])
        pltpu.async_copy(uring.at[0, slot],
                         stage_hbm.at[pl.ds(posj * D, D)], ssem)
        return ()

      lax.fori_loop(0, mcnt, extract, ())
      drain_n(mcnt, uring)

    tail_table(ulist_id, ulist_pos, ucnt, utail, ustage_hbm)
    tail_table(tlist_id, tlist_pos, tcnt, ttail, tstage_hbm)


def _dot_body(uid_hbm, tid_hbm, ustage_hbm, tstage_hbm, ubias_hbm, tbias_hbm,
              gbias_hbm, out_hbm,
              uid_v, tid_v, urows, trows, ub_v, tb_v, gb_v, out_v, sem, bsem):
  wid = lax.axis_index("s") * NC + lax.axis_index("c")
  base = wid * BPW

  pltpu.sync_copy(uid_hbm.at[pl.ds(base, BPW)], uid_v)
  pltpu.sync_copy(tid_hbm.at[pl.ds(base, BPW)], tid_v)
  pltpu.sync_copy(gbias_hbm, gb_v)

  bias_copies = []
  for j in range(NCH):
    sl = pl.ds(j * CH, CH)
    bias_copies.append(
        pltpu.async_copy(ubias_hbm.at[uid_v.at[sl]], ub_v.at[sl], bsem))
    bias_copies.append(
        pltpu.async_copy(tbias_hbm.at[tid_v.at[sl]], tb_v.at[sl], bsem))

  c_u = pltpu.async_copy(ustage_hbm.at[pl.ds(base * D, BPW * D)], urows, sem)
  c_t = pltpu.async_copy(tstage_hbm.at[pl.ds(base * D, BPW * D)], trows, sem)
  c_u.wait()
  c_t.wait()
  for bc in bias_copies:
    bc.wait()

  lanes = lax.iota(jnp.int32, 16)
  gb = gb_v[...]

  def group_body(g, _):
    elem0 = (g * 16 + lanes) * D
    acc = jnp.zeros((16,), jnp.float32)
    for j in range(D):
      idx = elem0 + jnp.bitwise_and(lanes + j, D - 1)
      u = plsc.load_gather(urows, [idx])
      t = plsc.load_gather(trows, [idx])
      acc = acc + u * t
    gsl = pl.ds(g * 16, 16)
    out_v[gsl] = acc + ub_v[gsl] + tb_v[gsl] + gb
    return ()

  lax.fori_loop(0, GROUPS, group_body, ())

  pltpu.sync_copy(out_v, out_hbm.at[pl.ds(base, BPW)])


@jax.jit
def _cf_call(uid, tid, uembT, tembT, utail_in, ttail_in, ubias, tbias, gbias):
  gather_kern = functools.partial(
      pl.kernel,
      out_type=(jax.ShapeDtypeStruct((BATCH * D,), jnp.float32),
                jax.ShapeDtypeStruct((BATCH * D,), jnp.float32)),
      mesh=_mesh(),
      compiler_params=_params,
      scratch_types=[
          pltpu.VMEM((BATCH // 4,), jnp.int32),
          pltpu.VMEM((LCAP + 16,), jnp.int32),
          pltpu.VMEM((LCAP + 16,), jnp.int32),
          pltpu.VMEM((LCAP + 16,), jnp.int32),
          pltpu.VMEM((LCAP + 16,), jnp.int32),
          pltpu.VMEM((8, SCAP), jnp.int32),
          pltpu.VMEM((8, SCAP), jnp.int32),
          pltpu.VMEM((8, SCAP), jnp.int32),
          pltpu.VMEM((8, SCAP), jnp.int32),
          pltpu.VMEM((4, D, CW), jnp.float32),
          pltpu.VMEM((4, D, CW), jnp.float32),
          pltpu.VMEM((D, 64), jnp.float32),
          pltpu.VMEM((D, 64), jnp.float32),
          pltpu.VMEM((MCAP + 16,), jnp.int32),
          pltpu.VMEM((MCAP + 16,), jnp.int32),
          pltpu.VMEM((4, MCAP, D), jnp.float32),
          pltpu.VMEM((4, MCAP, D), jnp.float32),
          pltpu.SMEM((32,), jnp.int32),
          pltpu.SemaphoreType.DMA,
          pltpu.SemaphoreType.DMA,
      ],
  )(_gather_body)
  ustage, tstage = gather_kern(uid, tid, uembT, tembT, utail_in, ttail_in)

  dot_kern = functools.partial(
      pl.kernel,
      out_type=jax.ShapeDtypeStruct((BATCH,), jnp.float32),
      mesh=_mesh(),
      compiler_params=_params,
      scratch_types=[
          pltpu.VMEM((BPW,), jnp.int32),
          pltpu.VMEM((BPW,), jnp.int32),
          pltpu.VMEM((BPW * D,), jnp.float32),
          pltpu.VMEM((BPW * D,), jnp.float32),
          pltpu.VMEM((BPW,), jnp.float32),
          pltpu.VMEM((BPW,), jnp.float32),
          pltpu.VMEM((16,), jnp.float32),
          pltpu.VMEM((BPW,), jnp.float32),
          pltpu.SemaphoreType.DMA,
          pltpu.SemaphoreType.DMA,
      ],
  )(_dot_body)
  return dot_kern(uid, tid, ustage, tstage, ubias, tbias, gbias)


def kernel(user_ids, track_ids, user_embeddings, track_embeddings,
           user_bias, track_bias, global_bias):
  uid = user_ids.astype(jnp.int32)
  tid = track_ids.astype(jnp.int32)
  uembT = user_embeddings.T
  tembT = track_embeddings.T
  utail_in = uembT[:, NFULL * 128:]
  ttail_in = tembT[:, NFULL * 128:]
  ubias = user_bias.reshape(-1)
  tbias = track_bias.reshape(-1)
  gbias = jnp.broadcast_to(global_bias, (16,))
  return _cf_call(uid, tid, uembT, tembT, utail_in, ttail_in,
                  ubias, tbias, gbias)


# speedup vs baseline: 1.0215x; 1.0215x over previous
import functools

import jax
import jax.numpy as jnp
from jax import lax
from jax.experimental import pallas as pl
from jax.experimental.pallas import tpu as pltpu
from jax.experimental.pallas import tpu_sc as plsc

BATCH = 16384
NROWS = 1000000
D = 64
NC = 2
NS = 16
NW = NC * NS
BPW = BATCH // NW
NBLK = 7813
NFULL = 7812
RPW = 246
CW = 128
LCAP = 1024
SCAP = 208
MCAP = 64
IDCH = BATCH // 16
CH = 128
NCH = BPW // CH
GROUPS = BPW // 16

_params = pltpu.CompilerParams(
    needs_layout_passes=False, use_tc_tiling_on_sc=True)
_mesh = lambda: plsc.VectorSubcoreMesh(core_axis_name="c", subcore_axis_name="s")


def _gather_body(uid_hbm, tid_hbm, uembT_hbm, tembT_hbm, utail_hbm,
                 ttail_hbm, ustage_hbm, tstage_hbm,
                 ids_v, ulist_id, ulist_pos, tlist_id, tlist_pos,
                 usub_id, usub_pos, tsub_id, tsub_pos,
                 ubuf, tbuf, utail, ttail, mb_id, mb_pos, ring,
                 subcnt, fsem, ssem):
  wid = lax.axis_index("s") * NC + lax.axis_index("c")
  lo_b = wid * RPW
  hi_b = jnp.minimum(lo_b + RPW, NFULL)
  nchunks = hi_b - lo_b

  lanes = lax.iota(jnp.int32, 16)

  def bin_ids(list_id, list_pos):
    def body(k, cnt):
      idv = ids_v[pl.ds(k * 16, 16)]
      blk = lax.shift_right_logical(idv, 7)
      m = jnp.logical_and(blk >= lo_b, blk < lo_b + RPW)
      plsc.store_compressed(list_id.at[pl.ds(cnt, 16)], idv, mask=m)
      plsc.store_compressed(list_pos.at[pl.ds(cnt, 16)],
                            k * 16 + lanes, mask=m)
      n = plsc.all_reduce_population_count(m)
      return cnt + n[0]
    return lax.fori_loop(0, IDCH, body, jnp.int32(0))

  pltpu.sync_copy(uid_hbm, ids_v)
  ucnt = bin_ids(ulist_id, ulist_pos)
  pltpu.sync_copy(tid_hbm, ids_v)
  tcnt = bin_ids(tlist_id, tlist_pos)

  def split_list(list_id, list_pos, sub_id, sub_pos, cnt, cbase):
    ngroups = lax.shift_right_logical(cnt + 15, 4)
    for sub in range(8):
      def body(g, scnt):
        gsl = pl.ds(g * 16, 16)
        idv = list_id[gsl]
        posv = list_pos[gsl]
        chunk = lax.shift_right_logical(idv, 7) - lo_b
        m = jnp.logical_and(lax.shift_right_logical(chunk, 5) == sub,
                            g * 16 + lanes < cnt)
        plsc.store_compressed(sub_id.at[sub, pl.ds(scnt, 16)], idv, mask=m)
        plsc.store_compressed(sub_pos.at[sub, pl.ds(scnt, 16)], posv, mask=m)
        n = plsc.all_reduce_population_count(m)
        return scnt + n[0]
      scnt = lax.fori_loop(0, ngroups, body, jnp.int32(0))
      subcnt[cbase + sub] = jnp.minimum(scnt, SCAP - 16)

  split_list(ulist_id, ulist_pos, usub_id, usub_pos, ucnt, 0)
  split_list(tlist_id, tlist_pos, tsub_id, tsub_pos, tcnt, 8)

  crange = [lanes + 16 * k for k in range(4)]

  def process(i, cb, buf, sub_id, sub_pos, cbase, stage_hbm):
    sub = lax.shift_right_logical(i, 5)
    scnt = subcnt[cbase + sub]
    ngroups = lax.shift_right_logical(scnt + 15, 4)

    def scan(g, mcnt):
      gsl = pl.ds(g * 16, 16)
      idv = sub_id[sub, gsl]
      posv = sub_pos[sub, gsl]
      chunk = lax.shift_right_logical(idv, 7) - lo_b
      m = jnp.logical_and(chunk == i, g * 16 + lanes < scnt)
      plsc.store_compressed(mb_id.at[pl.ds(mcnt, 16)], idv, mask=m)
      plsc.store_compressed(mb_pos.at[pl.ds(mcnt, 16)], posv, mask=m)
      n = plsc.all_reduce_population_count(m)
      return mcnt + n[0]

    mcnt = lax.fori_loop(0, ngroups, scan, jnp.int32(0))
    mcnt = jnp.minimum(mcnt, MCAP)

    def extract(j, _):
      idj = mb_id[pl.ds(j, 16)][0]
      posj = mb_pos[pl.ds(j, 16)][0]
      colw = jnp.broadcast_to(idj - cb, (16,))
      slot = jnp.bitwise_and(j, MCAP - 1)
      for k in range(4):
        ring[slot, pl.ds(k * 16, 16)] = plsc.load_gather(
            buf, [crange[k], colw])
      pltpu.async_copy(ring.at[slot],
                       stage_hbm.at[pl.ds(posj * D, D)], ssem)
      return ()

    lax.fori_loop(0, mcnt, extract, ())

    def drain(j, _):
      pltpu.make_async_copy(stage_hbm.at[pl.ds(0, D)], ring.at[0], ssem).wait()
      return ()

    lax.fori_loop(0, mcnt, drain, ())

  def fire(i, slot):
    off = pl.multiple_of((lo_b + i) * 128, 128)
    pltpu.async_copy(uembT_hbm.at[:, pl.ds(off, CW)], ubuf.at[slot], fsem)
    pltpu.async_copy(tembT_hbm.at[:, pl.ds(off, CW)], tbuf.at[slot], fsem)

  def wait_pair():
    pltpu.make_async_copy(uembT_hbm.at[:, pl.ds(0, CW)], ubuf.at[0],
                          fsem).wait()
    pltpu.make_async_copy(tembT_hbm.at[:, pl.ds(0, CW)], tbuf.at[0],
                          fsem).wait()

  for pre in range(3):
    fire(jnp.int32(pre), pre)

  def chunk_body(i, _):
    @pl.when(i < nchunks)
    def _():
      fire(i, jnp.bitwise_and(i, 3))

    wait_pair()
    pi = i - 3
    cb = (lo_b + pi) * 128
    pslot = jnp.bitwise_and(pi, 3)
    process(pi, cb, ubuf.at[pslot], usub_id, usub_pos, 0, ustage_hbm)
    process(pi, cb, tbuf.at[pslot], tsub_id, tsub_pos, 8, tstage_hbm)
    return ()

  lax.fori_loop(3, nchunks + 3, chunk_body, ())

  @pl.when(wid == NW - 1)
  def _():
    pltpu.async_copy(utail_hbm, utail, fsem)
    pltpu.async_copy(ttail_hbm, ttail, fsem)
    pltpu.make_async_copy(utail_hbm, utail, fsem).wait()
    pltpu.make_async_copy(ttail_hbm, ttail, fsem).wait()

    def tail_table(list_id, list_pos, cnt, buf, stage_hbm):
      ngroups = lax.shift_right_logical(cnt + 15, 4)

      def scan(g, mcnt):
        gsl = pl.ds(g * 16, 16)
        idv = list_id[gsl]
        posv = list_pos[gsl]
        m = jnp.logical_and(lax.shift_right_logical(idv, 7) == NFULL,
                            g * 16 + lanes < cnt)
        plsc.store_compressed(mb_id.at[pl.ds(mcnt, 16)], idv, mask=m)
        plsc.store_compressed(mb_pos.at[pl.ds(mcnt, 16)], posv, mask=m)
        n = plsc.all_reduce_population_count(m)
        return mcnt + n[0]

      mcnt = jnp.minimum(lax.fori_loop(0, ngroups, scan, jnp.int32(0)), MCAP)

      def extract(j, _):
        idj = mb_id[pl.ds(j, 16)][0]
        posj = mb_pos[pl.ds(j, 16)][0]
        colw = jnp.broadcast_to(jnp.bitwise_and(idj, 127), (16,))
        slot = jnp.bitwise_and(j, MCAP - 1)
        for k in range(4):
          ring[slot, pl.ds(k * 16, 16)] = plsc.load_gather(
              buf, [crange[k], colw])
        pltpu.async_copy(ring.at[slot],
                         stage_hbm.at[pl.ds(posj * D, D)], ssem)
        return ()

      lax.fori_loop(0, mcnt, extract, ())

      def drain(j, _):
        pltpu.make_async_copy(stage_hbm.at[pl.ds(0, D)], ring.at[0],
                              ssem).wait()
        return ()

      lax.fori_loop(0, mcnt, drain, ())

    tail_table(ulist_id, ulist_pos, ucnt, utail, ustage_hbm)
    tail_table(tlist_id, tlist_pos, tcnt, ttail, tstage_hbm)


def _dot_body(uid_hbm, tid_hbm, ustage_hbm, tstage_hbm, ubias_hbm, tbias_hbm,
              gbias_hbm, out_hbm,
              uid_v, tid_v, urows, trows, ub_v, tb_v, gb_v, out_v, sem, bsem):
  wid = lax.axis_index("s") * NC + lax.axis_index("c")
  base = wid * BPW

  pltpu.sync_copy(uid_hbm.at[pl.ds(base, BPW)], uid_v)
  pltpu.sync_copy(tid_hbm.at[pl.ds(base, BPW)], tid_v)
  pltpu.sync_copy(gbias_hbm, gb_v)

  bias_copies = []
  for j in range(NCH):
    sl = pl.ds(j * CH, CH)
    bias_copies.append(
        pltpu.async_copy(ubias_hbm.at[uid_v.at[sl]], ub_v.at[sl], bsem))
    bias_copies.append(
        pltpu.async_copy(tbias_hbm.at[tid_v.at[sl]], tb_v.at[sl], bsem))

  c_u = pltpu.async_copy(ustage_hbm.at[pl.ds(base * D, BPW * D)], urows, sem)
  c_t = pltpu.async_copy(tstage_hbm.at[pl.ds(base * D, BPW * D)], trows, sem)
  c_u.wait()
  c_t.wait()
  for bc in bias_copies:
    bc.wait()

  lanes = lax.iota(jnp.int32, 16)
  gb = gb_v[...]

  def group_body(g, _):
    elem0 = (g * 16 + lanes) * D
    acc = jnp.zeros((16,), jnp.float32)
    for j in range(D):
      idx = elem0 + jnp.bitwise_and(lanes + j, D - 1)
      u = plsc.load_gather(urows, [idx])
      t = plsc.load_gather(trows, [idx])
      acc = acc + u * t
    gsl = pl.ds(g * 16, 16)
    out_v[gsl] = acc + ub_v[gsl] + tb_v[gsl] + gb
    return ()

  lax.fori_loop(0, GROUPS, group_body, ())

  pltpu.sync_copy(out_v, out_hbm.at[pl.ds(base, BPW)])


@jax.jit
def _cf_call(uid, tid, uembT, tembT, utail_in, ttail_in, ubias, tbias, gbias):
  gather_kern = functools.partial(
      pl.kernel,
      out_type=(jax.ShapeDtypeStruct((BATCH * D,), jnp.float32),
                jax.ShapeDtypeStruct((BATCH * D,), jnp.float32)),
      mesh=_mesh(),
      compiler_params=_params,
      scratch_types=[
          pltpu.VMEM((BATCH,), jnp.int32),
          pltpu.VMEM((LCAP + 16,), jnp.int32),
          pltpu.VMEM((LCAP + 16,), jnp.int32),
          pltpu.VMEM((LCAP + 16,), jnp.int32),
          pltpu.VMEM((LCAP + 16,), jnp.int32),
          pltpu.VMEM((8, SCAP), jnp.int32),
          pltpu.VMEM((8, SCAP), jnp.int32),
          pltpu.VMEM((8, SCAP), jnp.int32),
          pltpu.VMEM((8, SCAP), jnp.int32),
          pltpu.VMEM((4, D, CW), jnp.float32),
          pltpu.VMEM((4, D, CW), jnp.float32),
          pltpu.VMEM((D, 64), jnp.float32),
          pltpu.VMEM((D, 64), jnp.float32),
          pltpu.VMEM((MCAP + 16,), jnp.int32),
          pltpu.VMEM((MCAP + 16,), jnp.int32),
          pltpu.VMEM((MCAP, D), jnp.float32),
          pltpu.SMEM((32,), jnp.int32),
          pltpu.SemaphoreType.DMA,
          pltpu.SemaphoreType.DMA,
      ],
  )(_gather_body)
  ustage, tstage = gather_kern(uid, tid, uembT, tembT, utail_in, ttail_in)

  dot_kern = functools.partial(
      pl.kernel,
      out_type=jax.ShapeDtypeStruct((BATCH,), jnp.float32),
      mesh=_mesh(),
      compiler_params=_params,
      scratch_types=[
          pltpu.VMEM((BPW,), jnp.int32),
          pltpu.VMEM((BPW,), jnp.int32),
          pltpu.VMEM((BPW * D,), jnp.float32),
          pltpu.VMEM((BPW * D,), jnp.float32),
          pltpu.VMEM((BPW,), jnp.float32),
          pltpu.VMEM((BPW,), jnp.float32),
          pltpu.VMEM((16,), jnp.float32),
          pltpu.VMEM((BPW,), jnp.float32),
          pltpu.SemaphoreType.DMA,
          pltpu.SemaphoreType.DMA,
      ],
  )(_dot_body)
  return dot_kern(uid, tid, ustage, tstage, ubias, tbias, gbias)


def kernel(user_ids, track_ids, user_embeddings, track_embeddings,
           user_bias, track_bias, global_bias):
  uid = user_ids.astype(jnp.int32)
  tid = track_ids.astype(jnp.int32)
  uembT = user_embeddings.T
  tembT = track_embeddings.T
  utail_in = uembT[:, NFULL * 128:]
  ttail_in = tembT[:, NFULL * 128:]
  ubias = user_bias.reshape(-1)
  tbias = track_bias.reshape(-1)
  gbias = jnp.broadcast_to(global_bias, (16,))
  return _cf_call(uid, tid, uembT, tembT, utail_in, ttail_in,
                  ubias, tbias, gbias)
